# Initial kernel scaffold; baseline (speedup 1.0000x reference)
import jax
import jax.numpy as jnp
from jax.experimental import pallas as pl


def kernel(img_lr, img_full, bases, wc1_w, wc1_b, wc2_w, wc2_b, wfc_w, wfc_b, rc1_w, rc1_b, rc2_w, rc2_b, fu_w, fu_b, fv_w, fv_b, fw_w, fw_b, fc_w, fc_b):
    raise NotImplementedError("write your pallas kernel here")


# speedup vs baseline: 1506.5070x; 1506.5070x over previous
import functools

import jax
import jax.numpy as jnp
from jax import lax
from jax.experimental import pallas as pl
from jax.experimental.pallas import tpu as pltpu
from jax.experimental.pallas import tpu_sc as plsc

G = 33
K = 8
R = 8
B = 16
H = 512
W = 512

NC = 2
NS = 16
NW = NC * NS

LUT_LEN = G * G * G * 3
LUT_PAD = ((LUT_LEN + 7) // 8) * 8

CR = 8
SEGS = W // 16
CHUNKS = (H // 2) // CR


def _conv2d(x, w, b, stride):
    y = lax.conv_general_dilated(x, w, (stride, stride), ((1, 1), (1, 1)),
                                 dimension_numbers=('NCHW', 'OIHW', 'NCHW'))
    return y + b[None, :, None, None]


def _encoder(x, w1, b1, w2, b2):
    h = jax.nn.relu(_conv2d(x, w1, b1, 2))
    h = jax.nn.relu(_conv2d(h, w2, b2, 2))
    return h.mean(axis=(2, 3))


def _lerp(a, b, t):
    return a + t * (b - a)


def _tri_body(lut_hbm, img_hbm, out_hbm, lut_v, rb, gb, bb):
    wid = lax.axis_index("s") * NC + lax.axis_index("c")
    img = wid // 2
    half = wid % 2
    pltpu.sync_copy(lut_hbm.at[img], lut_v)

    def chunk_body(ic, _):
        row0 = half * (H // 2) + ic * CR
        pltpu.sync_copy(img_hbm.at[img, 0, pl.ds(row0, CR), :], rb)
        pltpu.sync_copy(img_hbm.at[img, 1, pl.ds(row0, CR), :], gb)
        pltpu.sync_copy(img_hbm.at[img, 2, pl.ds(row0, CR), :], bb)

        def px_body(j, _):
            row = j >> 5
            seg = pl.multiple_of((j & (SEGS - 1)) << 4, 16)
            r = rb[row, pl.ds(seg, 16)]
            g = gb[row, pl.ds(seg, 16)]
            bl = bb[row, pl.ds(seg, 16)]
            hi = jnp.float32(G - 1 - 1e-6)
            x = jnp.clip(r * jnp.float32(G - 1), 0.0, hi)
            y = jnp.clip(g * jnp.float32(G - 1), 0.0, hi)
            z = jnp.clip(bl * jnp.float32(G - 1), 0.0, hi)
            x0 = x.astype(jnp.int32)
            y0 = y.astype(jnp.int32)
            z0 = z.astype(jnp.int32)
            xd = x - x0.astype(jnp.float32)
            yd = y - y0.astype(jnp.float32)
            zd = z - z0.astype(jnp.float32)
            x1 = jnp.minimum(x0 + 1, G - 1)
            y1 = jnp.minimum(y0 + 1, G - 1)
            z1 = jnp.minimum(z0 + 1, G - 1)
            xa = x0 * (3 * G * G)
            xb = x1 * (3 * G * G)
            ya = y0 * (3 * G)
            yb = y1 * (3 * G)
            za = z0 * 3
            zb = z1 * 3
            p00 = xa + ya
            p01 = xa + yb
            p10 = xb + ya
            p11 = xb + yb
            i000 = p00 + za
            i100 = p10 + za
            i010 = p01 + za
            i110 = p11 + za
            i001 = p00 + zb
            i101 = p10 + zb
            i011 = p01 + zb
            i111 = p11 + zb

            outs = []
            for ch in range(3):
                def gat(idx):
                    return plsc.load_gather(lut_v, [idx + ch] if ch else [idx])
                c000 = gat(i000)
                c100 = gat(i100)
                c010 = gat(i010)
                c110 = gat(i110)
                c001 = gat(i001)
                c101 = gat(i101)
                c011 = gat(i011)
                c111 = gat(i111)
                c00 = _lerp(c000, c100, xd)
                c10 = _lerp(c010, c110, xd)
                c01 = _lerp(c001, c101, xd)
                c11 = _lerp(c011, c111, xd)
                c0 = _lerp(c00, c10, yd)
                c1 = _lerp(c01, c11, yd)
                outs.append(_lerp(c0, c1, zd))

            rb[row, pl.ds(seg, 16)] = outs[0]
            gb[row, pl.ds(seg, 16)] = outs[1]
            bb[row, pl.ds(seg, 16)] = outs[2]
            return 0

        lax.fori_loop(0, CR * SEGS, px_body, 0)
        pltpu.sync_copy(rb, out_hbm.at[img, 0, pl.ds(row0, CR), :])
        pltpu.sync_copy(gb, out_hbm.at[img, 1, pl.ds(row0, CR), :])
        pltpu.sync_copy(bb, out_hbm.at[img, 2, pl.ds(row0, CR), :])
        return 0

    lax.fori_loop(0, CHUNKS, chunk_body, 0)


_tri_kernel = functools.partial(
    pl.kernel,
    mesh=plsc.VectorSubcoreMesh(core_axis_name="c", subcore_axis_name="s"),
    compiler_params=pltpu.CompilerParams(needs_layout_passes=False),
    out_type=jax.ShapeDtypeStruct((B, 3, H, W), jnp.float32),
    scratch_types=[
        pltpu.VMEM((LUT_PAD,), jnp.float32),
        pltpu.VMEM((CR, W), jnp.float32),
        pltpu.VMEM((CR, W), jnp.float32),
        pltpu.VMEM((CR, W), jnp.float32),
    ],
)(_tri_body)


def kernel(img_lr, img_full, bases, wc1_w, wc1_b, wc2_w, wc2_b, wfc_w, wfc_b,
           rc1_w, rc1_b, rc2_w, rc2_b, fu_w, fu_b, fv_w, fv_b, fw_w, fw_b,
           fc_w, fc_b):
    h_w = _encoder(img_lr, wc1_w, wc1_b, wc2_w, wc2_b)
    alpha = h_w @ wfc_w.T + wfc_b
    h_r = _encoder(img_lr, rc1_w, rc1_b, rc2_w, rc2_b)
    u = (h_r @ fu_w.T + fu_b).reshape(B, R, G)
    v = (h_r @ fv_w.T + fv_b).reshape(B, R, G)
    w = (h_r @ fw_w.T + fw_b).reshape(B, R, G)
    c = (h_r @ fc_w.T + fc_b).reshape(B, R, 3)
    core = u[:, :, :, None, None] * v[:, :, None, :, None] * w[:, :, None, None, :]
    delta = (core[..., None] * c[:, :, None, None, None, :]).sum(axis=1)
    L = (alpha.reshape(B, K, 1, 1, 1, 1) * bases[None]).sum(axis=1) + delta
    lut_flat = jnp.pad(L.reshape(B, LUT_LEN), ((0, 0), (0, LUT_PAD - LUT_LEN)))
    out = _tri_kernel(lut_flat, img_full)
    return (out, alpha, delta, L, jnp.abs(delta).mean())


# speedup vs baseline: 1859.5905x; 1.2344x over previous
import functools

import jax
import jax.numpy as jnp
from jax import lax
from jax.experimental import pallas as pl
from jax.experimental.pallas import tpu as pltpu
from jax.experimental.pallas import tpu_sc as plsc

G = 33
K = 8
R = 8
B = 16
H = 512
W = 512

NC = 2
NS = 16
NW = NC * NS

GCOL = G * G * 3
GPAD = 26 * 128
LUT_WORDS = G * GPAD

CR = 8
SEGS = W // 16
CHUNKS = (H // 2) // CR


def _conv2d(x, w, b, stride):
    y = lax.conv_general_dilated(x, w, (stride, stride), ((1, 1), (1, 1)),
                                 dimension_numbers=('NCHW', 'OIHW', 'NCHW'))
    return y + b[None, :, None, None]


def _encoder(x, w1, b1, w2, b2):
    h = jax.nn.relu(_conv2d(x, w1, b1, 2))
    h = jax.nn.relu(_conv2d(h, w2, b2, 2))
    return h.mean(axis=(2, 3))


def _lut_build_body(alpha_ref, ut_ref, vw_ref, bases_ref, delta_ref, l_ref,
                    psum_ref):
    ut = ut_ref[0]
    vw = vw_ref[0]
    delta = jnp.dot(ut, vw, preferred_element_type=jnp.float32)
    acc = alpha_ref[0, 0, 0] * bases_ref[0]
    for k in range(1, K):
        acc = acc + alpha_ref[0, 0, k] * bases_ref[k]
    delta_ref[0] = delta
    l_ref[0] = acc + delta
    psum_ref[0, 0, 0] = jnp.sum(jnp.abs(delta))


_lut_build = pl.pallas_call(
    _lut_build_body,
    grid=(B,),
    in_specs=[
        pl.BlockSpec((1, 1, K), lambda b: (b, 0, 0), memory_space=pltpu.SMEM),
        pl.BlockSpec((1, G, R), lambda b: (b, 0, 0)),
        pl.BlockSpec((1, R, GPAD), lambda b: (b, 0, 0)),
        pl.BlockSpec((K, G, GPAD), lambda b: (0, 0, 0)),
    ],
    out_specs=[
        pl.BlockSpec((1, G, GPAD), lambda b: (b, 0, 0)),
        pl.BlockSpec((1, G, GPAD), lambda b: (b, 0, 0)),
        pl.BlockSpec((1, 1, 1), lambda b: (b, 0, 0), memory_space=pltpu.SMEM),
    ],
    out_shape=[
        jax.ShapeDtypeStruct((B, G, GPAD), jnp.float32),
        jax.ShapeDtypeStruct((B, G, GPAD), jnp.float32),
        jax.ShapeDtypeStruct((B, 1, 1), jnp.float32),
    ],
)


def _lerp(a, b, t):
    return a + t * (b - a)


def _tri_body(lut_hbm, img_hbm, out_hbm, lut_v, rb, gb, bb):
    wid = lax.axis_index("s") * NC + lax.axis_index("c")
    img = wid // 2
    half = wid % 2
    pltpu.sync_copy(lut_hbm.at[img], lut_v)

    def chunk_body(ic, _):
        row0 = half * (H // 2) + ic * CR
        pltpu.sync_copy(img_hbm.at[img, 0, pl.ds(row0, CR), :], rb)
        pltpu.sync_copy(img_hbm.at[img, 1, pl.ds(row0, CR), :], gb)
        pltpu.sync_copy(img_hbm.at[img, 2, pl.ds(row0, CR), :], bb)

        def px_body(j, _):
            row = j >> 5
            seg = pl.multiple_of((j & (SEGS - 1)) << 4, 16)
            r = rb[row, pl.ds(seg, 16)]
            g = gb[row, pl.ds(seg, 16)]
            bl = bb[row, pl.ds(seg, 16)]
            hi = jnp.float32(G - 1 - 1e-6)
            x = jnp.clip(r * jnp.float32(G - 1), 0.0, hi)
            y = jnp.clip(g * jnp.float32(G - 1), 0.0, hi)
            z = jnp.clip(bl * jnp.float32(G - 1), 0.0, hi)
            x0 = x.astype(jnp.int32)
            y0 = y.astype(jnp.int32)
            z0 = z.astype(jnp.int32)
            xd = x - x0.astype(jnp.float32)
            yd = y - y0.astype(jnp.float32)
            zd = z - z0.astype(jnp.float32)
            x1 = jnp.minimum(x0 + 1, G - 1)
            y1 = jnp.minimum(y0 + 1, G - 1)
            z1 = jnp.minimum(z0 + 1, G - 1)
            xa = x0 * GPAD
            xb = x1 * GPAD
            ya = y0 * (3 * G)
            yb = y1 * (3 * G)
            za = z0 * 3
            zb = z1 * 3
            p00 = xa + ya
            p01 = xa + yb
            p10 = xb + ya
            p11 = xb + yb
            i000 = p00 + za
            i100 = p10 + za
            i010 = p01 + za
            i110 = p11 + za
            i001 = p00 + zb
            i101 = p10 + zb
            i011 = p01 + zb
            i111 = p11 + zb

            outs = []
            for ch in range(3):
                def gat(idx):
                    return plsc.load_gather(lut_v, [idx + ch] if ch else [idx])
                c000 = gat(i000)
                c100 = gat(i100)
                c010 = gat(i010)
                c110 = gat(i110)
                c001 = gat(i001)
                c101 = gat(i101)
                c011 = gat(i011)
                c111 = gat(i111)
                c00 = _lerp(c000, c100, xd)
                c10 = _lerp(c010, c110, xd)
                c01 = _lerp(c001, c101, xd)
                c11 = _lerp(c011, c111, xd)
                c0 = _lerp(c00, c10, yd)
                c1 = _lerp(c01, c11, yd)
                outs.append(_lerp(c0, c1, zd))

            rb[row, pl.ds(seg, 16)] = outs[0]
            gb[row, pl.ds(seg, 16)] = outs[1]
            bb[row, pl.ds(seg, 16)] = outs[2]
            return 0

        lax.fori_loop(0, CR * SEGS, px_body, 0)
        pltpu.sync_copy(rb, out_hbm.at[img, 0, pl.ds(row0, CR), :])
        pltpu.sync_copy(gb, out_hbm.at[img, 1, pl.ds(row0, CR), :])
        pltpu.sync_copy(bb, out_hbm.at[img, 2, pl.ds(row0, CR), :])
        return 0

    lax.fori_loop(0, CHUNKS, chunk_body, 0)


_tri_kernel = functools.partial(
    pl.kernel,
    mesh=plsc.VectorSubcoreMesh(core_axis_name="c", subcore_axis_name="s"),
    compiler_params=pltpu.CompilerParams(needs_layout_passes=False),
    out_type=jax.ShapeDtypeStruct((B, 3, H, W), jnp.float32),
    scratch_types=[
        pltpu.VMEM((LUT_WORDS,), jnp.float32),
        pltpu.VMEM((CR, W), jnp.float32),
        pltpu.VMEM((CR, W), jnp.float32),
        pltpu.VMEM((CR, W), jnp.float32),
    ],
)(_tri_body)


def kernel(img_lr, img_full, bases, wc1_w, wc1_b, wc2_w, wc2_b, wfc_w, wfc_b,
           rc1_w, rc1_b, rc2_w, rc2_b, fu_w, fu_b, fv_w, fv_b, fw_w, fw_b,
           fc_w, fc_b):
    h_w = _encoder(img_lr, wc1_w, wc1_b, wc2_w, wc2_b)
    alpha = h_w @ wfc_w.T + wfc_b
    h_r = _encoder(img_lr, rc1_w, rc1_b, rc2_w, rc2_b)
    u = (h_r @ fu_w.T + fu_b).reshape(B, R, G)
    v = (h_r @ fv_w.T + fv_b).reshape(B, R, G)
    w = (h_r @ fw_w.T + fw_b).reshape(B, R, G)
    c = (h_r @ fc_w.T + fc_b).reshape(B, R, 3)

    ut = jnp.transpose(u, (0, 2, 1))
    vw = (v[:, :, :, None, None] * w[:, :, None, :, None]
          * c[:, :, None, None, :]).reshape(B, R, GCOL)
    vw = jnp.pad(vw, ((0, 0), (0, 0), (0, GPAD - GCOL)))
    bases_p = jnp.pad(bases.reshape(K, G, GCOL),
                      ((0, 0), (0, 0), (0, GPAD - GCOL)))

    delta_p, l_p, psums = _lut_build(alpha.reshape(B, 1, K), ut, vw, bases_p)
    delta = delta_p[:, :, :GCOL].reshape(B, G, G, G, 3)
    L = l_p[:, :, :GCOL].reshape(B, G, G, G, 3)
    mean_abs = psums.sum() / jnp.float32(B * G * G * G * 3)

    out = _tri_kernel(l_p.reshape(B, LUT_WORDS), img_full)
    return (out, alpha, delta, L, mean_abs)


# speedup vs baseline: 1951.4084x; 1.0494x over previous
import functools

import jax
import jax.numpy as jnp
from jax import lax
from jax.experimental import pallas as pl
from jax.experimental.pallas import tpu as pltpu
from jax.experimental.pallas import tpu_sc as plsc

G = 33
K = 8
R = 8
B = 16
H = 512
W = 512

NC = 2
NS = 16
NW = NC * NS

GCOL = G * G * 3
GPAD = 26 * 128
LUT_WORDS = G * GPAD

CR = 8
SEGS = W // 16
CHUNKS = (H // 2) // CR


def _conv2d(x, w, b, stride):
    y = lax.conv_general_dilated(x, w, (stride, stride), ((1, 1), (1, 1)),
                                 dimension_numbers=('NCHW', 'OIHW', 'NCHW'))
    return y + b[None, :, None, None]


def _encoder(x, w1, b1, w2, b2):
    h = jax.nn.relu(_conv2d(x, w1, b1, 2))
    h = jax.nn.relu(_conv2d(h, w2, b2, 2))
    return h.mean(axis=(2, 3))


def _lut_build_body(alpha_ref, u_ref, vw_ref, bases_ref, delta_ref, l_ref,
                    lp_ref, psum_ref):
    u = u_ref[0]
    vw = vw_ref[0]
    delta = lax.dot_general(u, vw, (((0,), (0,)), ((), ())),
                            preferred_element_type=jnp.float32)
    acc = alpha_ref[0, 0, 0] * bases_ref[0]
    for k in range(1, K):
        acc = acc + alpha_ref[0, 0, k] * bases_ref[k]
    lut = acc + delta
    delta_ref[0] = delta
    l_ref[0] = lut
    lp_ref[0, :, :GCOL] = lut
    psum_ref[0, 0, 0] = jnp.sum(jnp.abs(delta))


_lut_build = pl.pallas_call(
    _lut_build_body,
    grid=(B,),
    in_specs=[
        pl.BlockSpec((1, 1, K), lambda b: (b, 0, 0), memory_space=pltpu.SMEM),
        pl.BlockSpec((1, R, G), lambda b: (b, 0, 0)),
        pl.BlockSpec((1, R, GCOL), lambda b: (b, 0, 0)),
        pl.BlockSpec((K, G, GCOL), lambda b: (0, 0, 0)),
    ],
    out_specs=[
        pl.BlockSpec((1, G, GCOL), lambda b: (b, 0, 0)),
        pl.BlockSpec((1, G, GCOL), lambda b: (b, 0, 0)),
        pl.BlockSpec((1, G, GPAD), lambda b: (b, 0, 0)),
        pl.BlockSpec((1, 1, 1), lambda b: (b, 0, 0), memory_space=pltpu.SMEM),
    ],
    out_shape=[
        jax.ShapeDtypeStruct((B, G, GCOL), jnp.float32),
        jax.ShapeDtypeStruct((B, G, GCOL), jnp.float32),
        jax.ShapeDtypeStruct((B, G, GPAD), jnp.float32),
        jax.ShapeDtypeStruct((B, 1, 1), jnp.float32),
    ],
)


def _lerp(a, b, t):
    return a + t * (b - a)


def _tri_body(lut_hbm, img_hbm, out_hbm, lut_v, rb, gb, bb):
    wid = lax.axis_index("s") * NC + lax.axis_index("c")
    img = wid // 2
    half = wid % 2
    pltpu.sync_copy(lut_hbm.at[img], lut_v)

    def chunk_body(ic, _):
        row0 = half * (H // 2) + ic * CR
        pltpu.sync_copy(img_hbm.at[img, 0, pl.ds(row0, CR), :], rb)
        pltpu.sync_copy(img_hbm.at[img, 1, pl.ds(row0, CR), :], gb)
        pltpu.sync_copy(img_hbm.at[img, 2, pl.ds(row0, CR), :], bb)

        def px_body(j, _):
            row = j >> 5
            seg = pl.multiple_of((j & (SEGS - 1)) << 4, 16)
            r = rb[row, pl.ds(seg, 16)]
            g = gb[row, pl.ds(seg, 16)]
            bl = bb[row, pl.ds(seg, 16)]
            hi = jnp.float32(G - 1 - 1e-6)
            x = jnp.clip(r * jnp.float32(G - 1), 0.0, hi)
            y = jnp.clip(g * jnp.float32(G - 1), 0.0, hi)
            z = jnp.clip(bl * jnp.float32(G - 1), 0.0, hi)
            x0 = x.astype(jnp.int32)
            y0 = y.astype(jnp.int32)
            z0 = z.astype(jnp.int32)
            xd = x - x0.astype(jnp.float32)
            yd = y - y0.astype(jnp.float32)
            zd = z - z0.astype(jnp.float32)
            x1 = jnp.minimum(x0 + 1, G - 1)
            y1 = jnp.minimum(y0 + 1, G - 1)
            z1 = jnp.minimum(z0 + 1, G - 1)
            xa = x0 * GPAD
            xb = x1 * GPAD
            ya = y0 * (3 * G)
            yb = y1 * (3 * G)
            za = z0 * 3
            zb = z1 * 3
            p00 = xa + ya
            p01 = xa + yb
            p10 = xb + ya
            p11 = xb + yb
            i000 = p00 + za
            i100 = p10 + za
            i010 = p01 + za
            i110 = p11 + za
            i001 = p00 + zb
            i101 = p10 + zb
            i011 = p01 + zb
            i111 = p11 + zb

            outs = []
            for ch in range(3):
                def gat(idx):
                    return plsc.load_gather(lut_v, [idx + ch] if ch else [idx])
                c000 = gat(i000)
                c100 = gat(i100)
                c010 = gat(i010)
                c110 = gat(i110)
                c001 = gat(i001)
                c101 = gat(i101)
                c011 = gat(i011)
                c111 = gat(i111)
                c00 = _lerp(c000, c100, xd)
                c10 = _lerp(c010, c110, xd)
                c01 = _lerp(c001, c101, xd)
                c11 = _lerp(c011, c111, xd)
                c0 = _lerp(c00, c10, yd)
                c1 = _lerp(c01, c11, yd)
                outs.append(_lerp(c0, c1, zd))

            rb[row, pl.ds(seg, 16)] = outs[0]
            gb[row, pl.ds(seg, 16)] = outs[1]
            bb[row, pl.ds(seg, 16)] = outs[2]
            return 0

        lax.fori_loop(0, CR * SEGS, px_body, 0)
        pltpu.sync_copy(rb, out_hbm.at[img, 0, pl.ds(row0, CR), :])
        pltpu.sync_copy(gb, out_hbm.at[img, 1, pl.ds(row0, CR), :])
        pltpu.sync_copy(bb, out_hbm.at[img, 2, pl.ds(row0, CR), :])
        return 0

    lax.fori_loop(0, CHUNKS, chunk_body, 0)


_tri_kernel = functools.partial(
    pl.kernel,
    mesh=plsc.VectorSubcoreMesh(core_axis_name="c", subcore_axis_name="s"),
    compiler_params=pltpu.CompilerParams(needs_layout_passes=False),
    out_type=jax.ShapeDtypeStruct((B, 3, H, W), jnp.float32),
    scratch_types=[
        pltpu.VMEM((LUT_WORDS,), jnp.float32),
        pltpu.VMEM((CR, W), jnp.float32),
        pltpu.VMEM((CR, W), jnp.float32),
        pltpu.VMEM((CR, W), jnp.float32),
    ],
)(_tri_body)


def kernel(img_lr, img_full, bases, wc1_w, wc1_b, wc2_w, wc2_b, wfc_w, wfc_b,
           rc1_w, rc1_b, rc2_w, rc2_b, fu_w, fu_b, fv_w, fv_b, fw_w, fw_b,
           fc_w, fc_b):
    h_w = _encoder(img_lr, wc1_w, wc1_b, wc2_w, wc2_b)
    alpha = h_w @ wfc_w.T + wfc_b
    h_r = _encoder(img_lr, rc1_w, rc1_b, rc2_w, rc2_b)
    u = (h_r @ fu_w.T + fu_b).reshape(B, R, G)
    v = (h_r @ fv_w.T + fv_b).reshape(B, R, G)
    w = (h_r @ fw_w.T + fw_b).reshape(B, R, G)
    c = (h_r @ fc_w.T + fc_b).reshape(B, R, 3)

    vw = (v[:, :, :, None, None] * w[:, :, None, :, None]
          * c[:, :, None, None, :]).reshape(B, R, GCOL)

    delta_e, l_e, l_p, psums = _lut_build(alpha.reshape(B, 1, K), u, vw,
                                          bases.reshape(K, G, GCOL))
    delta = delta_e.reshape(B, G, G, G, 3)
    L = l_e.reshape(B, G, G, G, 3)
    mean_abs = psums.sum() / jnp.float32(B * G * G * G * 3)

    out = _tri_kernel(l_p.reshape(B, LUT_WORDS), img_full)
    return (out, alpha, delta, L, mean_abs)


# speedup vs baseline: 2023.6426x; 1.0370x over previous
import functools

import jax
import jax.numpy as jnp
from jax import lax
from jax.experimental import pallas as pl
from jax.experimental.pallas import tpu as pltpu
from jax.experimental.pallas import tpu_sc as plsc

G = 33
K = 8
R = 8
B = 16
H = 512
W = 512

NC = 2
NS = 16
NW = NC * NS

GCOL = G * G * 3
GPAD = 26 * 128
LUT_WORDS = G * GPAD

CR = 8
SEGS = W // 16
CHUNKS = (H // 2) // CR


def _conv2d(x, w, b, stride):
    y = lax.conv_general_dilated(x, w, (stride, stride), ((1, 1), (1, 1)),
                                 dimension_numbers=('NCHW', 'OIHW', 'NCHW'))
    return y + b[None, :, None, None]


def _encoder(x, w1, b1, w2, b2):
    h = jax.nn.relu(_conv2d(x, w1, b1, 2))
    h = jax.nn.relu(_conv2d(h, w2, b2, 2))
    return h.mean(axis=(2, 3))


def _lut_build_body(alpha_ref, u_ref, vw_ref, bases_ref, delta_ref, l_ref,
                    lp_ref, psum_ref):
    u = u_ref[0]
    vw = vw_ref[0]
    delta = lax.dot_general(u, vw, (((0,), (0,)), ((), ())),
                            preferred_element_type=jnp.float32)
    acc = alpha_ref[0, 0, 0] * bases_ref[0]
    for k in range(1, K):
        acc = acc + alpha_ref[0, 0, k] * bases_ref[k]
    lut = acc + delta
    delta_ref[0] = delta
    l_ref[0] = lut
    lp_ref[0, :, :GCOL] = lut
    psum_ref[0, 0, 0] = jnp.sum(jnp.abs(delta))


_lut_build = pl.pallas_call(
    _lut_build_body,
    grid=(B,),
    in_specs=[
        pl.BlockSpec((1, 1, K), lambda b: (b, 0, 0), memory_space=pltpu.SMEM),
        pl.BlockSpec((1, R, G), lambda b: (b, 0, 0)),
        pl.BlockSpec((1, R, GCOL), lambda b: (b, 0, 0)),
        pl.BlockSpec((K, G, GCOL), lambda b: (0, 0, 0)),
    ],
    out_specs=[
        pl.BlockSpec((1, G, GCOL), lambda b: (b, 0, 0)),
        pl.BlockSpec((1, G, GCOL), lambda b: (b, 0, 0)),
        pl.BlockSpec((1, G, GPAD), lambda b: (b, 0, 0)),
        pl.BlockSpec((1, 1, 1), lambda b: (b, 0, 0), memory_space=pltpu.SMEM),
    ],
    out_shape=[
        jax.ShapeDtypeStruct((B, G, GCOL), jnp.float32),
        jax.ShapeDtypeStruct((B, G, GCOL), jnp.float32),
        jax.ShapeDtypeStruct((B, G, GPAD), jnp.float32),
        jax.ShapeDtypeStruct((B, 1, 1), jnp.float32),
    ],
)


def _lerp(a, b, t):
    return a + t * (b - a)


def _tri_body(lut_hbm, img_hbm, out_hbm, lut_v, buf):
    wid = lax.axis_index("s") * NC + lax.axis_index("c")
    img = wid // 2
    half = wid % 2
    pltpu.sync_copy(lut_hbm.at[img], lut_v)

    def chunk_body(ic, _):
        row0 = half * (H // 2) + ic * CR
        pltpu.sync_copy(img_hbm.at[img, :, pl.ds(row0, CR), :], buf)

        def px_body(j, _):
            row = j >> 5
            seg = pl.multiple_of((j & (SEGS - 1)) << 4, 16)
            r = buf[0, row, pl.ds(seg, 16)]
            g = buf[1, row, pl.ds(seg, 16)]
            bl = buf[2, row, pl.ds(seg, 16)]
            hi = jnp.float32(G - 1 - 1e-6)
            x = jnp.clip(r * jnp.float32(G - 1), 0.0, hi)
            y = jnp.clip(g * jnp.float32(G - 1), 0.0, hi)
            z = jnp.clip(bl * jnp.float32(G - 1), 0.0, hi)
            x0 = x.astype(jnp.int32)
            y0 = y.astype(jnp.int32)
            z0 = z.astype(jnp.int32)
            xd = x - x0.astype(jnp.float32)
            yd = y - y0.astype(jnp.float32)
            zd = z - z0.astype(jnp.float32)
            xa = x0 * GPAD
            ya = y0 * (3 * G)
            za = z0 * 3
            xb = jnp.minimum(xa + GPAD, (G - 1) * GPAD)
            yb = jnp.minimum(ya + 3 * G, (G - 1) * 3 * G)
            zb = jnp.minimum(za + 3, (G - 1) * 3)
            p00 = xa + ya
            p01 = xa + yb
            p10 = xb + ya
            p11 = xb + yb
            i000 = p00 + za
            i100 = p10 + za
            i010 = p01 + za
            i110 = p11 + za
            i001 = p00 + zb
            i101 = p10 + zb
            i011 = p01 + zb
            i111 = p11 + zb

            outs = []
            for ch in range(3):
                def gat(idx):
                    return plsc.load_gather(lut_v, [idx + ch] if ch else [idx])
                c000 = gat(i000)
                c100 = gat(i100)
                c010 = gat(i010)
                c110 = gat(i110)
                c001 = gat(i001)
                c101 = gat(i101)
                c011 = gat(i011)
                c111 = gat(i111)
                c00 = _lerp(c000, c100, xd)
                c10 = _lerp(c010, c110, xd)
                c01 = _lerp(c001, c101, xd)
                c11 = _lerp(c011, c111, xd)
                c0 = _lerp(c00, c10, yd)
                c1 = _lerp(c01, c11, yd)
                outs.append(_lerp(c0, c1, zd))

            buf[0, row, pl.ds(seg, 16)] = outs[0]
            buf[1, row, pl.ds(seg, 16)] = outs[1]
            buf[2, row, pl.ds(seg, 16)] = outs[2]
            return 0

        lax.fori_loop(0, CR * SEGS, px_body, 0)
        pltpu.sync_copy(buf, out_hbm.at[img, :, pl.ds(row0, CR), :])
        return 0

    lax.fori_loop(0, CHUNKS, chunk_body, 0)


_tri_kernel = functools.partial(
    pl.kernel,
    mesh=plsc.VectorSubcoreMesh(core_axis_name="c", subcore_axis_name="s"),
    compiler_params=pltpu.CompilerParams(needs_layout_passes=False),
    out_type=jax.ShapeDtypeStruct((B, 3, H, W), jnp.float32),
    scratch_types=[
        pltpu.VMEM((LUT_WORDS,), jnp.float32),
        pltpu.VMEM((3, CR, W), jnp.float32),
    ],
)(_tri_body)


def kernel(img_lr, img_full, bases, wc1_w, wc1_b, wc2_w, wc2_b, wfc_w, wfc_b,
           rc1_w, rc1_b, rc2_w, rc2_b, fu_w, fu_b, fv_w, fv_b, fw_w, fw_b,
           fc_w, fc_b):
    h_w = _encoder(img_lr, wc1_w, wc1_b, wc2_w, wc2_b)
    alpha = h_w @ wfc_w.T + wfc_b
    h_r = _encoder(img_lr, rc1_w, rc1_b, rc2_w, rc2_b)
    u = (h_r @ fu_w.T + fu_b).reshape(B, R, G)
    v = (h_r @ fv_w.T + fv_b).reshape(B, R, G)
    w = (h_r @ fw_w.T + fw_b).reshape(B, R, G)
    c = (h_r @ fc_w.T + fc_b).reshape(B, R, 3)

    vw = (v[:, :, :, None, None] * w[:, :, None, :, None]
          * c[:, :, None, None, :]).reshape(B, R, GCOL)

    delta_e, l_e, l_p, psums = _lut_build(alpha.reshape(B, 1, K), u, vw,
                                          bases.reshape(K, G, GCOL))
    one = lax.optimization_barrier(jnp.float32(1.0))
    delta = delta_e.reshape(B, G, G, G, 3) * one
    L = l_e.reshape(B, G, G, G, 3) * one
    mean_abs = psums.sum() / jnp.float32(B * G * G * G * 3)

    out = _tri_kernel(l_p.reshape(B, LUT_WORDS), img_full)
    return (out, alpha, delta, L, mean_abs)
